# baseline (speedup 1.0000x reference)
import jax
import jax.numpy as jnp
from jax import lax
from jax.experimental import pallas as pl
from jax.experimental.pallas import tpu as pltpu
from jax.experimental.pallas import tpu_sc as plsc

B = 4
S = 2048
NEMBD = 128
NW = 32
POS = S // NW
LANES = 16
NVEC = NEMBD // LANES


def _embd_body(idx_hbm, wte_hbm, wpe_hbm, out_hbm,
               idx_v, rows_v, wpe_v,
               sem_i, sem_w, sem_g, sem_o):
    c = lax.axis_index("c")
    s = lax.axis_index("s")
    wid = s * 2 + c
    pbase = wid * POS

    idx_copies = [
        pltpu.async_copy(idx_hbm.at[b, pl.ds(pbase, POS)], idx_v.at[b], sem_i)
        for b in range(B)
    ]
    wpe_copy = pltpu.async_copy(wpe_hbm.at[pl.ds(pbase, POS)], wpe_v, sem_w)
    for cp in idx_copies:
        cp.wait()

    gathers = [
        pltpu.async_copy(
            wte_hbm.at[idx_v.at[b]], rows_v.at[pl.ds(b * POS, POS)], sem_g
        )
        for b in range(B)
    ]
    wpe_copy.wait()
    for g in gathers:
        g.wait()

    def add_pos(p, carry):
        for j in range(NVEC):
            sl = pl.ds(j * LANES, LANES)
            w = wpe_v[p, sl]
            for b in range(B):
                plsc.addupdate(rows_v.at[b * POS + p, sl], w)
        return carry

    lax.fori_loop(0, POS, add_pos, 0)

    stores = [
        pltpu.async_copy(
            rows_v.at[pl.ds(b * POS, POS)],
            out_hbm.at[pl.ds(b * S + pbase, POS)],
            sem_o,
        )
        for b in range(B)
    ]
    for st in stores:
        st.wait()


@jax.jit
def _embd(idx, wte, wpe):
    mesh = plsc.VectorSubcoreMesh(core_axis_name="c", subcore_axis_name="s")
    return pl.kernel(
        _embd_body,
        out_type=jax.ShapeDtypeStruct((B * S, NEMBD), jnp.float32),
        mesh=mesh,
        scratch_types=[
            pltpu.VMEM((B, POS), jnp.int32),
            pltpu.VMEM((B * POS, NEMBD), jnp.float32),
            pltpu.VMEM((POS, NEMBD), jnp.float32),
        ] + [pltpu.SemaphoreType.DMA] * 4,
    )(idx, wte, wpe)


def kernel(idx, wte, wpe):
    out = _embd(idx.astype(jnp.int32), wte, wpe)
    return out.reshape(B, S, NEMBD)
